# Initial kernel scaffold; baseline (speedup 1.0000x reference)
import jax
import jax.numpy as jnp
from jax.experimental import pallas as pl


def kernel(hidden_states, mass, gate_W, mass_bias, W1, b1, W2, b2):
    raise NotImplementedError("write your pallas kernel here")


# speedup vs baseline: 3.5065x; 3.5065x over previous
import functools

import jax
import jax.numpy as jnp
from jax.experimental import pallas as pl
from jax.experimental.pallas import tpu as pltpu

B, T, D, E, DFF = 1, 2048, 1024, 8, 4096


def _router_body(x_ref, m_ref, gw_ref, mb_ref, w_ref, aux_ref):
    x = x_ref[...]
    gw = gw_ref[...]
    logits = jax.lax.dot_general(
        x, gw, (((1,), (1,)), ((), ())),
        preferred_element_type=jnp.float32)
    logits = logits + m_ref[...] * mb_ref[...]
    mx = jnp.max(logits, axis=1, keepdims=True)
    ex = jnp.exp(logits - mx)
    p = ex / jnp.sum(ex, axis=1, keepdims=True)

    eio = jax.lax.broadcasted_iota(jnp.int32, p.shape, 1)
    p1 = jnp.max(p, axis=1, keepdims=True)
    i1 = jnp.min(jnp.where(p == p1, eio, E), axis=1, keepdims=True)
    pm = jnp.where(eio == i1, -jnp.inf, p)
    p2 = jnp.max(pm, axis=1, keepdims=True)
    i2 = jnp.min(jnp.where(pm == p2, eio, E), axis=1, keepdims=True)
    w_ref[...] = jnp.where(eio == i1, p1, 0.0) + jnp.where(eio == i2, p2, 0.0)

    imp = jnp.sum(p, axis=0, keepdims=True)
    target = jnp.float32(T) / jnp.float32(E)
    aux_ref[...] = jnp.mean((imp - target) ** 2, keepdims=True).reshape(1, 1)


def _router(x, m, gate_W, mass_bias):
    w, aux = pl.pallas_call(
        _router_body,
        out_shape=(
            jax.ShapeDtypeStruct((T, E), jnp.float32),
            jax.ShapeDtypeStruct((1, 1), jnp.float32),
        ),
    )(x, m, gate_W, mass_bias.reshape(1, E))
    return w, aux[0, 0]


_TB = 1024
_FCH = 512


def _ffn_body(x_ref, w_ref, W1_ref, b1_ref, W2_ref, b2_ref, out_ref):
    e = pl.program_id(1)
    f = pl.program_id(2)

    eio = jax.lax.broadcasted_iota(jnp.int32, (w_ref.shape[0], E), 1)
    wcol = jnp.sum(jnp.where(eio == e, w_ref[...], 0.0), axis=1,
                   keepdims=True)

    @pl.when((e == 0) & (f == 0))
    def _init():
        out_ref[...] = jnp.zeros_like(out_ref)

    h = jax.lax.dot_general(
        x_ref[...], W1_ref[0], (((1,), (1,)), ((), ())),
        preferred_element_type=jnp.float32)
    h = h + b1_ref[0]
    h = 0.5 * h * (1.0 + jax.lax.erf(h * 0.7071067811865476))
    hw = h * wcol
    acc = jax.lax.dot_general(
        hw, W2_ref[0], (((1,), (1,)), ((), ())),
        preferred_element_type=jnp.float32)

    @pl.when(f == 0)
    def _bias2():
        out_ref[...] += wcol * b2_ref[0]

    out_ref[...] += acc


def _ffn(x, w, W1, b1, W2, b2):
    grid = (T // _TB, E, DFF // _FCH)
    return pl.pallas_call(
        _ffn_body,
        grid=grid,
        in_specs=[
            pl.BlockSpec((_TB, D), lambda t, e, f: (t, 0)),
            pl.BlockSpec((_TB, E), lambda t, e, f: (t, 0)),
            pl.BlockSpec((1, _FCH, D), lambda t, e, f: (e, f, 0)),
            pl.BlockSpec((1, 1, _FCH),
                         lambda t, e, f: (e * (DFF // _FCH) + f, 0, 0)),
            pl.BlockSpec((1, D, _FCH), lambda t, e, f: (e, 0, f)),
            pl.BlockSpec((1, 1, D), lambda t, e, f: (e, 0, 0)),
        ],
        out_specs=pl.BlockSpec((_TB, D), lambda t, e, f: (t, 0)),
        out_shape=jax.ShapeDtypeStruct((T, D), jnp.float32),
    )(x, w, W1, b1.reshape(E * (DFF // _FCH), 1, _FCH), W2,
      b2.reshape(E, 1, D))


def kernel(hidden_states, mass, gate_W, mass_bias, W1, b1, W2, b2):
    x = hidden_states.reshape(T, D)
    m = mass.reshape(T, 1)
    w, aux = _router(x, m, gate_W, mass_bias)
    out = _ffn(x, w, W1, b1, W2, b2)
    return out.reshape(hidden_states.shape), aux
